# baseline (speedup 1.0000x reference)
import functools

import jax
import jax.numpy as jnp
from jax import lax
from jax.experimental import pallas as pl
from jax.experimental.pallas import tpu as pltpu
from jax.experimental.pallas import tpu_sc as plsc

_B = 16384
_D = 32
_NW = 32
_BPW = _B // _NW
_CH = 4
_CB = _BPW // _CH

_V = 1000000
_WIN = 512
_NFULL = _V // _WIN
_TAIL_OUT = (_NFULL * _WIN) // 4


def _take16(v, idx):
    dnums = lax.GatherDimensionNumbers(
        offset_dims=(), collapsed_slice_dims=(0,), start_index_map=(0,))
    return lax.gather(v, idx.reshape(16, 1), dnums, (1,),
                      mode=lax.GatherScatterMode.PROMISE_IN_BOUNDS)


def _relayout_body(utT_hbm, itT_hbm, utail_hbm, itail_hbm, uo_hbm, io_hbm,
                   win_v, trows_v, tail_v, sem):
    wid = lax.axis_index("s") * 2 + lax.axis_index("c")
    lo = wid * 61 + jnp.minimum(wid, 1)
    cnt = 61 + jnp.where(wid < 1, 1, 0)
    iota = lax.iota(jnp.int32, 16)
    row_lo = iota
    row_hi = iota + 16

    def transpose_window():
        def o_loop(o, carry):
            for k in range(8):
                rows = row_lo if (k % 2) == 0 else row_hi
                cols = jnp.full((16,), 4 * o + (k // 2), jnp.int32)
                trows_v[o, pl.ds(16 * k, 16)] = plsc.load_gather(
                    win_v, [rows, cols])
            return carry

        lax.fori_loop(0, 128, o_loop, 0)

    def do_table(src, tail, dst):
        def w_loop(w, carry):
            off = pl.multiple_of(w * _WIN, _WIN)
            pltpu.sync_copy(src.at[:, pl.ds(off, _WIN)], win_v)
            transpose_window()
            pltpu.sync_copy(trows_v,
                            dst.at[pl.ds(pl.multiple_of(w * 128, 128), 128)])
            return carry

        lax.fori_loop(lo, lo + cnt, w_loop, 0)

        @pl.when(wid == _NW - 1)
        def _():
            pltpu.sync_copy(tail, tail_v)
            pltpu.sync_copy(tail_v, dst.at[pl.ds(_TAIL_OUT, 16)])

    do_table(utT_hbm, utail_hbm, uo_hbm)
    do_table(itT_hbm, itail_hbm, io_hbm)


def _gmf_body(uq_hbm, iq_hbm, us_hbm, is_hbm, par_hbm, utab_hbm, itab_hbm,
              out_hbm, uq_v, iq_v, us_v, is_v, urows_v, irows_v, par_v,
              out_v, sem):
    wid = lax.axis_index("s") * 2 + lax.axis_index("c")

    pltpu.sync_copy(uq_hbm.at[pl.ds(wid * _CH, _CH)], uq_v)
    pltpu.sync_copy(iq_hbm.at[pl.ds(wid * _CH, _CH)], iq_v)
    pltpu.sync_copy(us_hbm.at[pl.ds(wid * _CH, _CH)], us_v)
    pltpu.sync_copy(is_hbm.at[pl.ds(wid * _CH, _CH)], is_v)
    pltpu.sync_copy(par_hbm, par_v)

    iota = lax.iota(jnp.int32, 16)
    neg_b = par_v[pl.ds(_D, 16)]
    w_lo = par_v[pl.ds(0, 16)]
    w_hi = par_v[pl.ds(16, 16)]
    fifteen = jnp.full((16,), 15, jnp.int32)

    for c in range(_CH):
        cu = pltpu.async_copy(utab_hbm.at[uq_v.at[c]], urows_v, sem)
        ci = pltpu.async_copy(itab_hbm.at[iq_v.at[c]], irows_v, sem)
        cu.wait()
        ci.wait()

        def row_group(rg, carry):
            base = rg * 16
            acc = neg_b
            for j in range(16):
                r = base + j
                jj = jnp.full((16,), r, jnp.int32)
                sub = plsc.load_gather(us_v.at[c], [jj])
                col_lo = sub + iota
                col_hi = col_lo + 16
                u_lo = plsc.load_gather(urows_v, [jj, col_lo])
                u_hi = plsc.load_gather(urows_v, [jj, col_hi])
                sub_i = plsc.load_gather(is_v.at[c], [jj])
                icol_lo = sub_i + iota
                i_lo = plsc.load_gather(irows_v, [jj, icol_lo])
                i_hi = plsc.load_gather(irows_v, [jj, icol_lo + 16])
                s = u_lo * i_lo * w_lo + u_hi * i_hi * w_hi
                hs = _take16(plsc.cumsum(s), fifteen)
                acc = jnp.where(iota == j, hs + neg_b, acc)
            out_v[pl.ds(c * _CB + base, 16)] = 1.0 / (1.0 + jnp.exp(acc))
            return carry

        lax.fori_loop(0, _CB // 16, row_group, 0)

    pltpu.sync_copy(out_v, out_hbm.at[pl.ds(wid * _BPW, _BPW)])


def kernel(user_indices, item_indices, user_table, item_table, W, b):
    uidx = user_indices.astype(jnp.int32)
    iidx = item_indices.astype(jnp.int32)
    uq = (uidx >> 2).reshape(_NW * _CH, _CB)
    iq = (iidx >> 2).reshape(_NW * _CH, _CB)
    us = ((uidx & 3) * _D).reshape(_NW * _CH, _CB)
    i_s = ((iidx & 3) * _D).reshape(_NW * _CH, _CB)
    params = jnp.concatenate(
        [-W.reshape(_D), jnp.broadcast_to(-b, (16,))]).astype(jnp.float32)

    mesh = plsc.VectorSubcoreMesh(core_axis_name="c", subcore_axis_name="s")

    relayout = functools.partial(
        pl.kernel, mesh=mesh,
        compiler_params=pltpu.CompilerParams(needs_layout_passes=False),
        out_type=(jax.ShapeDtypeStruct((_V // 4, 128), jnp.float32),
                  jax.ShapeDtypeStruct((_V // 4, 128), jnp.float32)),
        scratch_types=[
            pltpu.VMEM((_D, _WIN), jnp.float32),
            pltpu.VMEM((128, 128), jnp.float32),
            pltpu.VMEM((16, 128), jnp.float32),
            pltpu.SemaphoreType.DMA,
        ],
    )(_relayout_body)
    utail = user_table[_NFULL * _WIN:, :].reshape(16, 128)
    itail = item_table[_NFULL * _WIN:, :].reshape(16, 128)
    ut2, it2 = relayout(user_table.T, item_table.T, utail, itail)

    run = functools.partial(
        pl.kernel, mesh=mesh,
        compiler_params=pltpu.CompilerParams(needs_layout_passes=False),
        out_type=jax.ShapeDtypeStruct((_B,), jnp.float32),
        scratch_types=[
            pltpu.VMEM((_CH, _CB), jnp.int32),
            pltpu.VMEM((_CH, _CB), jnp.int32),
            pltpu.VMEM((_CH, _CB), jnp.int32),
            pltpu.VMEM((_CH, _CB), jnp.int32),
            pltpu.VMEM((_CB, 128), jnp.float32),
            pltpu.VMEM((_CB, 128), jnp.float32),
            pltpu.VMEM((_D + 16,), jnp.float32),
            pltpu.VMEM((_BPW,), jnp.float32),
            pltpu.SemaphoreType.DMA,
        ],
    )(_gmf_body)
    out = run(uq, iq, us, i_s, params, ut2, it2)
    return out.reshape(_B, 1)


# speedup vs baseline: 2.0485x; 2.0485x over previous
import functools

import jax
import jax.numpy as jnp
from jax import lax
from jax.experimental import pallas as pl
from jax.experimental.pallas import tpu as pltpu
from jax.experimental.pallas import tpu_sc as plsc

_B = 16384
_D = 32
_NW = 32
_BPW = _B // _NW
_CH = 4
_CB = _BPW // _CH


def _take16(v, idx):
    dnums = lax.GatherDimensionNumbers(
        offset_dims=(), collapsed_slice_dims=(0,), start_index_map=(0,))
    return lax.gather(v, idx.reshape(16, 1), dnums, (1,),
                      mode=lax.GatherScatterMode.PROMISE_IN_BOUNDS)


def _gmf_body(uidx_hbm, iidx_hbm, par_hbm, utab_hbm, itab_hbm, out_hbm,
              uidx_v, iidx_v, urows_v, irows_v, par_v, out_v, sem):
    wid = lax.axis_index("s") * 2 + lax.axis_index("c")

    pltpu.sync_copy(uidx_hbm.at[pl.ds(wid * _CH, _CH)], uidx_v)
    pltpu.sync_copy(iidx_hbm.at[pl.ds(wid * _CH, _CH)], iidx_v)
    pltpu.sync_copy(par_hbm, par_v)

    copies = []
    for k in range(_CH):
        copies.append(pltpu.async_copy(
            utab_hbm.at[uidx_v.at[k]], urows_v.at[pl.ds(k * _CB, _CB)], sem))
        copies.append(pltpu.async_copy(
            itab_hbm.at[iidx_v.at[k]], irows_v.at[pl.ds(k * _CB, _CB)], sem))
    for c in copies:
        c.wait()

    iota = lax.iota(jnp.int32, 16)
    neg_b = par_v[pl.ds(_D, 16)]
    w_lo = par_v[pl.ds(0, 16)]
    w_hi = par_v[pl.ds(16, 16)]
    fifteen = jnp.full((16,), 15, jnp.int32)

    def row_group(rg, carry):
        base = rg * 16
        acc = neg_b
        for j in range(16):
            r = base + j
            u_lo = urows_v[r, pl.ds(0, 16)]
            u_hi = urows_v[r, pl.ds(16, 16)]
            i_lo = irows_v[r, pl.ds(0, 16)]
            i_hi = irows_v[r, pl.ds(16, 16)]
            s = u_lo * i_lo * w_lo + u_hi * i_hi * w_hi
            hs = _take16(plsc.cumsum(s), fifteen)
            acc = jnp.where(iota == j, hs + neg_b, acc)
        out_v[pl.ds(base, 16)] = 1.0 / (1.0 + jnp.exp(acc))
        return carry

    lax.fori_loop(0, _BPW // 16, row_group, 0)

    pltpu.sync_copy(out_v, out_hbm.at[pl.ds(wid * _BPW, _BPW)])


def kernel(user_indices, item_indices, user_table, item_table, W, b):
    uidx = user_indices.astype(jnp.int32).reshape(_NW * _CH, _CB)
    iidx = item_indices.astype(jnp.int32).reshape(_NW * _CH, _CB)
    params = jnp.concatenate(
        [-W.reshape(_D), jnp.broadcast_to(-b, (16,))]).astype(jnp.float32)

    mesh = plsc.VectorSubcoreMesh(core_axis_name="c", subcore_axis_name="s")
    run = functools.partial(
        pl.kernel, mesh=mesh,
        compiler_params=pltpu.CompilerParams(
            needs_layout_passes=False, use_tc_tiling_on_sc=False),
        out_type=jax.ShapeDtypeStruct((_B,), jnp.float32),
        scratch_types=[
            pltpu.VMEM((_CH, _CB), jnp.int32),
            pltpu.VMEM((_CH, _CB), jnp.int32),
            pltpu.VMEM((_BPW, _D), jnp.float32),
            pltpu.VMEM((_BPW, _D), jnp.float32),
            pltpu.VMEM((_D + 16,), jnp.float32),
            pltpu.VMEM((_BPW,), jnp.float32),
            pltpu.SemaphoreType.DMA,
        ],
    )(_gmf_body)
    out = run(uidx, iidx, params, user_table, item_table)
    return out.reshape(_B, 1)
